# baseline (speedup 1.0000x reference)
import functools

import jax
import jax.numpy as jnp
from jax import lax
from jax.experimental import pallas as pl
from jax.experimental.pallas import tpu as pltpu

_THRESHOLD = 0.35
_NEGPOS_RATIO = 7
_VAR0 = 0.1
_VAR1 = 0.2
_SMOOTH_POINT = 0.2


def _body(loc_ref, conf_ref, iou_ref, pri_ref, tgt_ref, out_ref,
          bto_ref, mx1_ref, my1_ref, mx2_ref, my2_ref,
          vm_ref, vc_ref, v_ref, *, T, S, P):
    f32 = jnp.float32
    i32 = jnp.int32
    NC = S // 8

    txs = []
    areas = []
    for t in range(T):
        tx1 = tgt_ref[0, t, 0]
        ty1 = tgt_ref[0, t, 1]
        tx2 = tgt_ref[0, t, 2]
        ty2 = tgt_ref[0, t, 3]
        txs.append((tx1, ty1, tx2, ty2))
        areas.append((tx2 - tx1) * (ty2 - ty1))

    vm_ref[...] = jnp.full((T, 8, 128), -1.0, f32)
    vc_ref[...] = jnp.zeros((T, 8, 128), i32)

    def chunk_match(i, _):
        rows = pl.ds(i * 8, 8)
        ppx1 = pri_ref[0, rows, :]
        ppy1 = pri_ref[1, rows, :]
        ppx2 = pri_ref[2, rows, :]
        ppy2 = pri_ref[3, rows, :]
        area_b = pri_ref[4, rows, :]
        bto = None
        for t in range(T):
            tx1, ty1, tx2, ty2 = txs[t]
            ix1 = jnp.maximum(ppx1, tx1)
            iy1 = jnp.maximum(ppy1, ty1)
            ix2 = jnp.minimum(ppx2, tx2)
            iy2 = jnp.minimum(ppy2, ty2)
            iw = jnp.maximum(ix2 - ix1, 0.0)
            ih = jnp.maximum(iy2 - iy1, 0.0)
            inter = iw * ih
            union = (areas[t] + area_b) - inter
            ov = inter / union
            if t == 0:
                bto = ov
                mx1 = jnp.full((8, 128), tx1, f32)
                my1 = jnp.full((8, 128), ty1, f32)
                mx2 = jnp.full((8, 128), tx2, f32)
                my2 = jnp.full((8, 128), ty2, f32)
            else:
                upd = ov > bto
                bto = jnp.where(upd, ov, bto)
                mx1 = jnp.where(upd, tx1, mx1)
                my1 = jnp.where(upd, ty1, my1)
                mx2 = jnp.where(upd, tx2, mx2)
                my2 = jnp.where(upd, ty2, my2)
            pvm = vm_ref[t]
            upd2 = ov > pvm
            vm_ref[t] = jnp.where(upd2, ov, pvm)
            vc_ref[t] = jnp.where(upd2, i, vc_ref[t])
        bto_ref[rows, :] = bto
        mx1_ref[rows, :] = mx1
        my1_ref[rows, :] = my1
        mx2_ref[rows, :] = mx2
        my2_ref[rows, :] = my2
        return 0

    lax.fori_loop(0, NC, chunk_match, 0)

    sub8 = lax.broadcasted_iota(i32, (8, 128), 0)
    lane8 = lax.broadcasted_iota(i32, (8, 128), 1)
    bpi = []
    for t in range(T):
        vm = vm_ref[t]
        m = jnp.max(vm)
        gidx = (vc_ref[t] * 8 + sub8) * 128 + lane8
        bpi.append(jnp.min(jnp.where(vm == m, gidx, P)))

    li = lax.broadcasted_iota(i32, (1, 128), 1)
    for t in range(T):
        r = bpi[t] // 128
        c = bpi[t] % 128
        hit = li == c
        tx1, ty1, tx2, ty2 = txs[t]
        bto_ref[pl.ds(r, 1), :] = jnp.where(hit, 2.0, bto_ref[pl.ds(r, 1), :])
        mx1_ref[pl.ds(r, 1), :] = jnp.where(hit, tx1, mx1_ref[pl.ds(r, 1), :])
        my1_ref[pl.ds(r, 1), :] = jnp.where(hit, ty1, my1_ref[pl.ds(r, 1), :])
        mx2_ref[pl.ds(r, 1), :] = jnp.where(hit, tx2, mx2_ref[pl.ds(r, 1), :])
        my2_ref[pl.ds(r, 1), :] = jnp.where(hit, ty2, my2_ref[pl.ds(r, 1), :])

    def chunk_loss(i, acc):
        a_l, a_iou, a_c, a_np = acc
        rows = pl.ds(i * 8, 8)
        bto = bto_ref[rows, :]
        mx1 = mx1_ref[rows, :]
        my1 = my1_ref[rows, :]
        mx2 = mx2_ref[rows, :]
        my2 = my2_ref[rows, :]
        pcx = pri_ref[5, rows, :]
        pcy = pri_ref[6, rows, :]
        pw = pri_ref[7, rows, :]
        ph = pri_ref[8, rows, :]
        v0w = pri_ref[9, rows, :]
        v0h = pri_ref[10, rows, :]

        pos = bto >= _THRESHOLD
        posf = pos.astype(f32)

        g_cx = ((mx1 + mx2) * 0.5 - pcx) / v0w
        g_cy = ((my1 + my2) * 0.5 - pcy) / v0h
        g_w = jnp.log((mx2 - mx1) / pw) / _VAR1
        g_h = jnp.log((my2 - my1) / ph) / _VAR1

        px1 = jnp.where(pos, loc_ref[0, 0, rows, :], 0.0)
        py1 = jnp.where(pos, loc_ref[1, 0, rows, :], 0.0)
        px2 = jnp.where(pos, loc_ref[2, 0, rows, :], 1.0)
        py2 = jnp.where(pos, loc_ref[3, 0, rows, :], 1.0)
        qx1 = jnp.where(pos, g_cx, 0.0)
        qy1 = jnp.where(pos, g_cy, 0.0)
        qx2 = jnp.where(pos, g_w, 1.0)
        qy2 = jnp.where(pos, g_h, 1.0)
        ex1 = jnp.minimum(px1, qx1)
        ey1 = jnp.minimum(py1, qy1)
        ix1 = jnp.maximum(px1, qx1)
        iy1 = jnp.maximum(py1, qy1)
        ix2 = jnp.minimum(px2, qx2)
        iy2 = jnp.minimum(py2, qy2)
        xmin = jnp.minimum(ix1, ix2)
        ymin = jnp.minimum(iy1, iy2)
        xmax = jnp.maximum(ix1, ix2)
        ymax = jnp.maximum(iy1, iy2)
        inter_e = ((ix2 - ex1) * (iy2 - ey1) + (xmin - ex1) * (ymin - ey1)
                   - (ix1 - ex1) * (ymax - ey1) - (xmax - ex1) * (iy1 - ey1))
        union_e = ((px2 - px1) * (py2 - py1)
                   + (qx2 - qx1) * (qy2 - qy1) - inter_e)
        iou_e = inter_e / (union_e + 1e-12)
        ious = 1.0 - iou_e
        el = jnp.where(ious < _SMOOTH_POINT,
                       0.5 * ious * ious / _SMOOTH_POINT,
                       ious - 0.5 * _SMOOTH_POINT)

        d = iou_ref[0, rows, :] - bto
        ad = jnp.abs(d)
        sl1 = jnp.where(ad < 1.0, 0.5 * d * d, ad - 0.5)

        c0 = conf_ref[0, 0, rows, :]
        c1 = conf_ref[1, 0, rows, :]
        mc = jnp.maximum(c0, c1)
        lse = mc + jnp.log(jnp.exp(c0 - mc) + jnp.exp(c1 - mc))
        gath = jnp.where(pos, c1, c0)
        lca = lse - gath
        v_ref[rows, :] = jnp.maximum(jnp.where(pos, 0.0, lca), 0.0)

        return (a_l + el * posf, a_iou + sl1 * posf,
                a_c + lca * posf, a_np + pos.astype(i32))

    zf = jnp.zeros((8, 128), f32)
    acc = lax.fori_loop(0, NC, chunk_loss,
                        (zf, zf, zf, jnp.zeros((8, 128), i32)))
    loss_l = jnp.sum(acc[0])
    loss_iou = jnp.sum(acc[1])
    loss_c_pos = jnp.sum(acc[2])
    npos_i = jnp.sum(acc[3])
    npos_f = npos_i.astype(f32)

    v = v_ref[...]
    bits = lax.bitcast_convert_type(v, i32)
    k = jnp.minimum(_NEGPOS_RATIO * npos_i, P - 1)

    def bs(_, lohi):
        lo, hi = lohi
        mid = lo + (hi - lo) // 2
        cnt = jnp.sum((bits > mid).astype(i32))
        pred = cnt < k
        nlo = jnp.where(pred, lo, mid + 1)
        nhi = jnp.where(pred, mid, hi)
        live = lo < hi
        return (jnp.where(live, nlo, lo), jnp.where(live, nhi, hi))

    tau_bits, _ = lax.fori_loop(0, 31, bs, (jnp.int32(0), jnp.max(bits)))
    tau = jnp.max(jnp.where(bits == tau_bits, v, 0.0))
    cgt = jnp.sum((bits > tau_bits).astype(i32))
    sgt = jnp.sum(jnp.where(bits > tau_bits, v, 0.0))
    topk = sgt + tau * (k - cgt).astype(f32)
    topk = jnp.where(k > 0, topk, 0.0)
    loss_c = loss_c_pos + topk

    lo = lax.broadcasted_iota(i32, (1, 128), 1)
    row = (jnp.where(lo == 0, loss_l, 0.0)
           + jnp.where(lo == 1, loss_c, 0.0)
           + jnp.where(lo == 2, loss_iou, 0.0)
           + jnp.where(lo == 3, npos_f, 0.0))
    out_ref[...] = row[None]


@jax.jit
def kernel(loc_data, conf_data, iou_data, priors, targets):
    B, P, C = conf_data.shape
    T = targets.shape[1]
    S = P // 128

    lpl = jnp.transpose(loc_data, (2, 0, 1)).reshape(4, B, S, 128)
    cpl = jnp.transpose(conf_data, (2, 0, 1)).reshape(C, B, S, 128)
    ipl = iou_data.reshape(B, S, 128)

    pcx, pcy, pw, ph = (priors[:, 0], priors[:, 1], priors[:, 2], priors[:, 3])
    ppx1 = pcx - pw * 0.5
    ppy1 = pcy - ph * 0.5
    ppx2 = pcx + pw * 0.5
    ppy2 = pcy + ph * 0.5
    area_b = (ppx2 - ppx1) * (ppy2 - ppy1)
    ppl = jnp.stack([ppx1, ppy1, ppx2, ppy2, area_b,
                     pcx, pcy, pw, ph, _VAR0 * pw, _VAR0 * ph]).reshape(
                         11, S, 128)

    body = functools.partial(_body, T=T, S=S, P=P)
    out = pl.pallas_call(
        body,
        grid=(B,),
        in_specs=[
            pl.BlockSpec((4, 1, S, 128), lambda b: (0, b, 0, 0)),
            pl.BlockSpec((C, 1, S, 128), lambda b: (0, b, 0, 0)),
            pl.BlockSpec((1, S, 128), lambda b: (b, 0, 0)),
            pl.BlockSpec((11, S, 128), lambda b: (0, 0, 0)),
            pl.BlockSpec((1, T, 15), lambda b: (b, 0, 0),
                         memory_space=pltpu.SMEM),
        ],
        out_specs=pl.BlockSpec((1, 1, 128), lambda b: (b, 0, 0)),
        out_shape=jax.ShapeDtypeStruct((B, 1, 128), jnp.float32),
        scratch_shapes=[pltpu.VMEM((S, 128), jnp.float32)] * 5
        + [pltpu.VMEM((T, 8, 128), jnp.float32),
           pltpu.VMEM((T, 8, 128), jnp.int32),
           pltpu.VMEM((S, 128), jnp.float32)],
        compiler_params=pltpu.CompilerParams(
            dimension_semantics=("arbitrary",)),
    )(lpl, cpl, ipl, ppl, targets)

    s = jnp.sum(out[:, 0, :4], axis=0)
    n = jnp.maximum(s[3], 1.0)
    return (s[0] / n, s[1] / n, s[2] / n)


# speedup vs baseline: 1.1312x; 1.1312x over previous
import functools

import jax
import jax.numpy as jnp
from jax import lax
from jax.experimental import pallas as pl
from jax.experimental.pallas import tpu as pltpu

_THRESHOLD = 0.35
_NEGPOS_RATIO = 7
_VAR0 = 0.1
_VAR1 = 0.2
_SMOOTH_POINT = 0.2


def _body(loc_ref, conf_ref, iou_ref, pri_ref, tgt_ref, out_ref,
          bto_ref, mx1_ref, my1_ref, mx2_ref, my2_ref, *, T, S, P, BB):
    for bb in range(BB):
        _one_batch(loc_ref, conf_ref, iou_ref, pri_ref, tgt_ref, out_ref,
                   bto_ref, mx1_ref, my1_ref, mx2_ref, my2_ref,
                   T=T, S=S, P=P, bb=bb)


def _one_batch(loc_ref, conf_ref, iou_ref, pri_ref, tgt_ref, out_ref,
               bto_ref, mx1_ref, my1_ref, mx2_ref, my2_ref, *, T, S, P, bb):
    f32 = jnp.float32
    i32 = jnp.int32

    ppx1 = pri_ref[0]
    ppy1 = pri_ref[1]
    ppx2 = pri_ref[2]
    ppy2 = pri_ref[3]
    area_b = pri_ref[4]
    pcx = pri_ref[5]
    pcy = pri_ref[6]
    pw = pri_ref[7]
    ph = pri_ref[8]
    v0w = pri_ref[9]
    v0h = pri_ref[10]

    sub = lax.broadcasted_iota(i32, (S, 128), 0)
    lane = lax.broadcasted_iota(i32, (S, 128), 1)
    gidx = sub * 128 + lane

    bto = None
    bpi = []
    txs = []
    for t in range(T):
        tx1 = tgt_ref[bb, t, 0]
        ty1 = tgt_ref[bb, t, 1]
        tx2 = tgt_ref[bb, t, 2]
        ty2 = tgt_ref[bb, t, 3]
        txs.append((tx1, ty1, tx2, ty2))
        area_a = (tx2 - tx1) * (ty2 - ty1)
        ix1 = jnp.maximum(ppx1, tx1)
        iy1 = jnp.maximum(ppy1, ty1)
        ix2 = jnp.minimum(ppx2, tx2)
        iy2 = jnp.minimum(ppy2, ty2)
        iw = jnp.maximum(ix2 - ix1, 0.0)
        ih = jnp.maximum(iy2 - iy1, 0.0)
        inter = iw * ih
        union = (area_a + area_b) - inter
        ov = inter / union
        if t == 0:
            bto = ov
            mx1 = jnp.full((S, 128), tx1, f32)
            my1 = jnp.full((S, 128), ty1, f32)
            mx2 = jnp.full((S, 128), tx2, f32)
            my2 = jnp.full((S, 128), ty2, f32)
        else:
            upd = ov > bto
            bto = jnp.where(upd, ov, bto)
            mx1 = jnp.where(upd, tx1, mx1)
            my1 = jnp.where(upd, ty1, my1)
            mx2 = jnp.where(upd, tx2, mx2)
            my2 = jnp.where(upd, ty2, my2)
        m = jnp.max(ov)
        bpi.append(jnp.min(jnp.where(ov == m, gidx, P)))

    bto_ref[...] = bto
    mx1_ref[...] = mx1
    my1_ref[...] = my1
    mx2_ref[...] = mx2
    my2_ref[...] = my2

    li = lax.broadcasted_iota(i32, (1, 128), 1)
    for t in range(T):
        r = bpi[t] // 128
        c = bpi[t] % 128
        hit = li == c
        tx1, ty1, tx2, ty2 = txs[t]
        bto_ref[pl.ds(r, 1), :] = jnp.where(hit, 2.0, bto_ref[pl.ds(r, 1), :])
        mx1_ref[pl.ds(r, 1), :] = jnp.where(hit, tx1, mx1_ref[pl.ds(r, 1), :])
        my1_ref[pl.ds(r, 1), :] = jnp.where(hit, ty1, my1_ref[pl.ds(r, 1), :])
        mx2_ref[pl.ds(r, 1), :] = jnp.where(hit, tx2, mx2_ref[pl.ds(r, 1), :])
        my2_ref[pl.ds(r, 1), :] = jnp.where(hit, ty2, my2_ref[pl.ds(r, 1), :])

    bto = bto_ref[...]
    mx1 = mx1_ref[...]
    my1 = my1_ref[...]
    mx2 = mx2_ref[...]
    my2 = my2_ref[...]

    pos = bto >= _THRESHOLD
    posf = pos.astype(f32)
    npos_i = jnp.sum(pos, dtype=i32)
    npos_f = npos_i.astype(f32)

    g_cx = ((mx1 + mx2) * 0.5 - pcx) / v0w
    g_cy = ((my1 + my2) * 0.5 - pcy) / v0h
    g_w = jnp.log((mx2 - mx1) / pw) / _VAR1
    g_h = jnp.log((my2 - my1) / ph) / _VAR1

    px1 = jnp.where(pos, loc_ref[0, bb], 0.0)
    py1 = jnp.where(pos, loc_ref[1, bb], 0.0)
    px2 = jnp.where(pos, loc_ref[2, bb], 1.0)
    py2 = jnp.where(pos, loc_ref[3, bb], 1.0)
    qx1 = jnp.where(pos, g_cx, 0.0)
    qy1 = jnp.where(pos, g_cy, 0.0)
    qx2 = jnp.where(pos, g_w, 1.0)
    qy2 = jnp.where(pos, g_h, 1.0)
    ex1 = jnp.minimum(px1, qx1)
    ey1 = jnp.minimum(py1, qy1)
    ix1 = jnp.maximum(px1, qx1)
    iy1 = jnp.maximum(py1, qy1)
    ix2 = jnp.minimum(px2, qx2)
    iy2 = jnp.minimum(py2, qy2)
    xmin = jnp.minimum(ix1, ix2)
    ymin = jnp.minimum(iy1, iy2)
    xmax = jnp.maximum(ix1, ix2)
    ymax = jnp.maximum(iy1, iy2)
    inter_e = ((ix2 - ex1) * (iy2 - ey1) + (xmin - ex1) * (ymin - ey1)
               - (ix1 - ex1) * (ymax - ey1) - (xmax - ex1) * (iy1 - ey1))
    union_e = (px2 - px1) * (py2 - py1) + (qx2 - qx1) * (qy2 - qy1) - inter_e
    iou_e = inter_e / (union_e + 1e-12)
    ious = 1.0 - iou_e
    el = jnp.where(ious < _SMOOTH_POINT,
                   0.5 * ious * ious / _SMOOTH_POINT,
                   ious - 0.5 * _SMOOTH_POINT)
    loss_l = jnp.sum(el * posf)

    d = iou_ref[bb] - bto
    ad = jnp.abs(d)
    sl1 = jnp.where(ad < 1.0, 0.5 * d * d, ad - 0.5)
    loss_iou = jnp.sum(sl1 * posf)

    c0 = conf_ref[0, bb]
    c1 = conf_ref[1, bb]
    mc = jnp.maximum(c0, c1)
    lse = mc + jnp.log(jnp.exp(c0 - mc) + jnp.exp(c1 - mc))
    gath = jnp.where(pos, c1, c0)
    lca = lse - gath
    loss_c_pos = jnp.sum(lca * posf)

    v = jnp.maximum(jnp.where(pos, 0.0, lca), 0.0)
    bits = lax.bitcast_convert_type(v, i32)
    k = jnp.minimum(_NEGPOS_RATIO * npos_i, P - 1)

    def bs(_, lohi):
        lo, hi = lohi
        mid = lo + (hi - lo) // 2
        cnt = jnp.sum((bits > mid).astype(i32))
        pred = cnt < k
        nlo = jnp.where(pred, lo, mid + 1)
        nhi = jnp.where(pred, mid, hi)
        live = lo < hi
        return (jnp.where(live, nlo, lo), jnp.where(live, nhi, hi))

    tau_bits, _ = lax.fori_loop(0, 31, bs, (jnp.int32(0), jnp.max(bits)))
    tau = jnp.max(jnp.where(bits == tau_bits, v, 0.0))
    cgt = jnp.sum((bits > tau_bits).astype(i32))
    sgt = jnp.sum(jnp.where(bits > tau_bits, v, 0.0))
    topk = sgt + tau * (k - cgt).astype(f32)
    topk = jnp.where(k > 0, topk, 0.0)
    loss_c = loss_c_pos + topk

    lo = lax.broadcasted_iota(i32, (1, 128), 1)
    row = (jnp.where(lo == 0, loss_l, 0.0)
           + jnp.where(lo == 1, loss_c, 0.0)
           + jnp.where(lo == 2, loss_iou, 0.0)
           + jnp.where(lo == 3, npos_f, 0.0))
    out_ref[pl.ds(bb, 1)] = row[None]


@jax.jit
def kernel(loc_data, conf_data, iou_data, priors, targets):
    B, P, C = conf_data.shape
    T = targets.shape[1]
    S = P // 128

    lpl = jnp.transpose(loc_data, (2, 0, 1)).reshape(4, B, S, 128)
    cpl = jnp.transpose(conf_data, (2, 0, 1)).reshape(C, B, S, 128)
    ipl = iou_data.reshape(B, S, 128)

    pcx, pcy, pw, ph = (priors[:, 0], priors[:, 1], priors[:, 2], priors[:, 3])
    ppx1 = pcx - pw * 0.5
    ppy1 = pcy - ph * 0.5
    ppx2 = pcx + pw * 0.5
    ppy2 = pcy + ph * 0.5
    area_b = (ppx2 - ppx1) * (ppy2 - ppy1)
    ppl = jnp.stack([ppx1, ppy1, ppx2, ppy2, area_b,
                     pcx, pcy, pw, ph, _VAR0 * pw, _VAR0 * ph]).reshape(
                         11, S, 128)

    BB = 2
    body = functools.partial(_body, T=T, S=S, P=P, BB=BB)
    out = pl.pallas_call(
        body,
        grid=(B // BB,),
        in_specs=[
            pl.BlockSpec((4, BB, S, 128), lambda b: (0, b, 0, 0)),
            pl.BlockSpec((C, BB, S, 128), lambda b: (0, b, 0, 0)),
            pl.BlockSpec((BB, S, 128), lambda b: (b, 0, 0)),
            pl.BlockSpec((11, S, 128), lambda b: (0, 0, 0)),
            pl.BlockSpec((BB, T, 15), lambda b: (b, 0, 0),
                         memory_space=pltpu.SMEM),
        ],
        out_specs=pl.BlockSpec((BB, 1, 128), lambda b: (b, 0, 0)),
        out_shape=jax.ShapeDtypeStruct((B, 1, 128), jnp.float32),
        scratch_shapes=[pltpu.VMEM((S, 128), jnp.float32)] * 5,
        compiler_params=pltpu.CompilerParams(
            dimension_semantics=("arbitrary",)),
    )(lpl, cpl, ipl, ppl, targets)

    s = jnp.sum(out[:, 0, :4], axis=0)
    n = jnp.maximum(s[3], 1.0)
    return (s[0] / n, s[1] / n, s[2] / n)


# speedup vs baseline: 1.1595x; 1.0251x over previous
import functools

import jax
import jax.numpy as jnp
from jax import lax
from jax.experimental import pallas as pl
from jax.experimental.pallas import tpu as pltpu

_THRESHOLD = 0.35
_NEGPOS_RATIO = 7
_VAR0 = 0.1
_VAR1 = 0.2
_SMOOTH_POINT = 0.2


def _body(loc_ref, conf_ref, iou_ref, pri_ref, tgt_ref, out_ref,
          bto_ref, mx1_ref, my1_ref, mx2_ref, my2_ref, v_ref,
          *, T, S, P, BB):
    for bb in range(BB):
        _one_batch(loc_ref, conf_ref, iou_ref, pri_ref, tgt_ref, out_ref,
                   bto_ref, mx1_ref, my1_ref, mx2_ref, my2_ref, v_ref,
                   T=T, S=S, P=P, bb=bb)


def _one_batch(loc_ref, conf_ref, iou_ref, pri_ref, tgt_ref, out_ref,
               bto_ref, mx1_ref, my1_ref, mx2_ref, my2_ref, v_ref,
               *, T, S, P, bb):
    f32 = jnp.float32
    i32 = jnp.int32

    ppx1 = pri_ref[0]
    ppy1 = pri_ref[1]
    ppx2 = pri_ref[2]
    ppy2 = pri_ref[3]
    area_b = pri_ref[4]

    sub = lax.broadcasted_iota(i32, (S, 128), 0)
    lane = lax.broadcasted_iota(i32, (S, 128), 1)
    gidx = sub * 128 + lane

    bto = None
    bpi = []
    txs = []
    for t in range(T):
        tx1 = tgt_ref[bb, t, 0]
        ty1 = tgt_ref[bb, t, 1]
        tx2 = tgt_ref[bb, t, 2]
        ty2 = tgt_ref[bb, t, 3]
        txs.append((tx1, ty1, tx2, ty2))
        area_a = (tx2 - tx1) * (ty2 - ty1)
        ix1 = jnp.maximum(ppx1, tx1)
        iy1 = jnp.maximum(ppy1, ty1)
        ix2 = jnp.minimum(ppx2, tx2)
        iy2 = jnp.minimum(ppy2, ty2)
        iw = jnp.maximum(ix2 - ix1, 0.0)
        ih = jnp.maximum(iy2 - iy1, 0.0)
        inter = iw * ih
        union = (area_a + area_b) - inter
        ov = inter / union
        if t == 0:
            bto = ov
            mx1 = jnp.full((S, 128), tx1, f32)
            my1 = jnp.full((S, 128), ty1, f32)
            mx2 = jnp.full((S, 128), tx2, f32)
            my2 = jnp.full((S, 128), ty2, f32)
        else:
            upd = ov > bto
            bto = jnp.where(upd, ov, bto)
            mx1 = jnp.where(upd, tx1, mx1)
            my1 = jnp.where(upd, ty1, my1)
            mx2 = jnp.where(upd, tx2, mx2)
            my2 = jnp.where(upd, ty2, my2)
        m = jnp.max(ov)
        bpi.append(jnp.min(jnp.where(ov == m, gidx, P)))

    bto_ref[...] = bto
    mx1_ref[...] = mx1
    my1_ref[...] = my1
    mx2_ref[...] = mx2
    my2_ref[...] = my2

    li = lax.broadcasted_iota(i32, (1, 128), 1)
    for t in range(T):
        r = bpi[t] // 128
        c = bpi[t] % 128
        hit = li == c
        tx1, ty1, tx2, ty2 = txs[t]
        bto_ref[pl.ds(r, 1), :] = jnp.where(hit, 2.0, bto_ref[pl.ds(r, 1), :])
        mx1_ref[pl.ds(r, 1), :] = jnp.where(hit, tx1, mx1_ref[pl.ds(r, 1), :])
        my1_ref[pl.ds(r, 1), :] = jnp.where(hit, ty1, my1_ref[pl.ds(r, 1), :])
        mx2_ref[pl.ds(r, 1), :] = jnp.where(hit, tx2, mx2_ref[pl.ds(r, 1), :])
        my2_ref[pl.ds(r, 1), :] = jnp.where(hit, ty2, my2_ref[pl.ds(r, 1), :])

    CH = 64
    NC = S // CH

    def chunk_loss(i, acc):
        a_l, a_iou, a_c, a_np = acc
        rows = pl.ds(i * CH, CH)
        bto = bto_ref[rows, :]
        mx1 = mx1_ref[rows, :]
        my1 = my1_ref[rows, :]
        mx2 = mx2_ref[rows, :]
        my2 = my2_ref[rows, :]

        pos = bto >= _THRESHOLD
        posf = pos.astype(f32)

        g_cx = ((mx1 + mx2) * 0.5 - pri_ref[5, rows, :]) / pri_ref[9, rows, :]
        g_cy = ((my1 + my2) * 0.5 - pri_ref[6, rows, :]) / pri_ref[10, rows, :]
        g_w = jnp.log((mx2 - mx1) / pri_ref[7, rows, :]) / _VAR1
        g_h = jnp.log((my2 - my1) / pri_ref[8, rows, :]) / _VAR1

        px1 = jnp.where(pos, loc_ref[0, bb, rows, :], 0.0)
        py1 = jnp.where(pos, loc_ref[1, bb, rows, :], 0.0)
        px2 = jnp.where(pos, loc_ref[2, bb, rows, :], 1.0)
        py2 = jnp.where(pos, loc_ref[3, bb, rows, :], 1.0)
        qx1 = jnp.where(pos, g_cx, 0.0)
        qy1 = jnp.where(pos, g_cy, 0.0)
        qx2 = jnp.where(pos, g_w, 1.0)
        qy2 = jnp.where(pos, g_h, 1.0)
        ex1 = jnp.minimum(px1, qx1)
        ey1 = jnp.minimum(py1, qy1)
        ix1 = jnp.maximum(px1, qx1)
        iy1 = jnp.maximum(py1, qy1)
        ix2 = jnp.minimum(px2, qx2)
        iy2 = jnp.minimum(py2, qy2)
        xmin = jnp.minimum(ix1, ix2)
        ymin = jnp.minimum(iy1, iy2)
        xmax = jnp.maximum(ix1, ix2)
        ymax = jnp.maximum(iy1, iy2)
        inter_e = ((ix2 - ex1) * (iy2 - ey1) + (xmin - ex1) * (ymin - ey1)
                   - (ix1 - ex1) * (ymax - ey1) - (xmax - ex1) * (iy1 - ey1))
        union_e = ((px2 - px1) * (py2 - py1)
                   + (qx2 - qx1) * (qy2 - qy1) - inter_e)
        iou_e = inter_e / (union_e + 1e-12)
        ious = 1.0 - iou_e
        el = jnp.where(ious < _SMOOTH_POINT,
                       0.5 * ious * ious / _SMOOTH_POINT,
                       ious - 0.5 * _SMOOTH_POINT)

        d = iou_ref[bb, rows, :] - bto
        ad = jnp.abs(d)
        sl1 = jnp.where(ad < 1.0, 0.5 * d * d, ad - 0.5)

        c0 = conf_ref[0, bb, rows, :]
        c1 = conf_ref[1, bb, rows, :]
        mc = jnp.maximum(c0, c1)
        lse = mc + jnp.log(jnp.exp(c0 - mc) + jnp.exp(c1 - mc))
        gath = jnp.where(pos, c1, c0)
        lca = lse - gath
        v_ref[rows, :] = jnp.maximum(jnp.where(pos, 0.0, lca), 0.0)

        return (a_l + el * posf, a_iou + sl1 * posf,
                a_c + lca * posf, a_np + pos.astype(i32))

    zf = jnp.zeros((CH, 128), f32)
    acc = lax.fori_loop(0, NC, chunk_loss,
                        (zf, zf, zf, jnp.zeros((CH, 128), i32)))
    loss_l = jnp.sum(acc[0])
    loss_iou = jnp.sum(acc[1])
    loss_c_pos = jnp.sum(acc[2])
    npos_i = jnp.sum(acc[3])
    npos_f = npos_i.astype(f32)

    v = v_ref[...]
    bits = lax.bitcast_convert_type(v, i32)
    k = jnp.minimum(_NEGPOS_RATIO * npos_i, P - 1)

    def bs(_, lohi):
        lo, hi = lohi
        mid = lo + (hi - lo) // 2
        cnt = jnp.sum((bits > mid).astype(i32))
        pred = cnt < k
        nlo = jnp.where(pred, lo, mid + 1)
        nhi = jnp.where(pred, mid, hi)
        live = lo < hi
        return (jnp.where(live, nlo, lo), jnp.where(live, nhi, hi))

    tau_bits, _ = lax.fori_loop(0, 31, bs, (jnp.int32(0), jnp.max(bits)))
    tau = jnp.max(jnp.where(bits == tau_bits, v, 0.0))
    cgt = jnp.sum((bits > tau_bits).astype(i32))
    sgt = jnp.sum(jnp.where(bits > tau_bits, v, 0.0))
    topk = sgt + tau * (k - cgt).astype(f32)
    topk = jnp.where(k > 0, topk, 0.0)
    loss_c = loss_c_pos + topk

    lo = lax.broadcasted_iota(i32, (1, 128), 1)
    row = (jnp.where(lo == 0, loss_l, 0.0)
           + jnp.where(lo == 1, loss_c, 0.0)
           + jnp.where(lo == 2, loss_iou, 0.0)
           + jnp.where(lo == 3, npos_f, 0.0))
    out_ref[pl.ds(bb, 1)] = row[None]


@jax.jit
def kernel(loc_data, conf_data, iou_data, priors, targets):
    B, P, C = conf_data.shape
    T = targets.shape[1]
    S = P // 128

    lpl = jnp.transpose(loc_data, (2, 0, 1)).reshape(4, B, S, 128)
    cpl = jnp.transpose(conf_data, (2, 0, 1)).reshape(C, B, S, 128)
    ipl = iou_data.reshape(B, S, 128)

    pcx, pcy, pw, ph = (priors[:, 0], priors[:, 1], priors[:, 2], priors[:, 3])
    ppx1 = pcx - pw * 0.5
    ppy1 = pcy - ph * 0.5
    ppx2 = pcx + pw * 0.5
    ppy2 = pcy + ph * 0.5
    area_b = (ppx2 - ppx1) * (ppy2 - ppy1)
    ppl = jnp.stack([ppx1, ppy1, ppx2, ppy2, area_b,
                     pcx, pcy, pw, ph, _VAR0 * pw, _VAR0 * ph]).reshape(
                         11, S, 128)

    BB = 2
    body = functools.partial(_body, T=T, S=S, P=P, BB=BB)
    out = pl.pallas_call(
        body,
        grid=(B // BB,),
        in_specs=[
            pl.BlockSpec((4, BB, S, 128), lambda b: (0, b, 0, 0)),
            pl.BlockSpec((C, BB, S, 128), lambda b: (0, b, 0, 0)),
            pl.BlockSpec((BB, S, 128), lambda b: (b, 0, 0)),
            pl.BlockSpec((11, S, 128), lambda b: (0, 0, 0)),
            pl.BlockSpec((BB, T, 15), lambda b: (b, 0, 0),
                         memory_space=pltpu.SMEM),
        ],
        out_specs=pl.BlockSpec((BB, 1, 128), lambda b: (b, 0, 0)),
        out_shape=jax.ShapeDtypeStruct((B, 1, 128), jnp.float32),
        scratch_shapes=[pltpu.VMEM((S, 128), jnp.float32)] * 6,
        compiler_params=pltpu.CompilerParams(
            dimension_semantics=("arbitrary",)),
    )(lpl, cpl, ipl, ppl, targets)

    s = jnp.sum(out[:, 0, :4], axis=0)
    n = jnp.maximum(s[3], 1.0)
    return (s[0] / n, s[1] / n, s[2] / n)
